# Initial kernel scaffold; baseline (speedup 1.0000x reference)
import functools

import jax
import jax.numpy as jnp
from jax import lax
from jax.experimental import pallas as pl
from jax.experimental.pallas import tpu as pltpu
from jax.experimental.pallas import tpu_sc as plsc

NN = 10000
EE = 320000
NGG = 100
DD = 128

NS = 2 * NN
NP = 20480
ET = 2 * EE
NTILE = 16
NCORE = 2
TE = 40960
EP = NTILE * TE
CH = TE // 128
RPT = NP // NTILE
BLK = 1024
GRID = NP // BLK

_PREC = lax.Precision.HIGHEST


def _sc_mesh():
    return plsc.VectorSubcoreMesh(
        core_axis_name="c", subcore_axis_name="s",
        num_cores=NCORE, num_subcores=NTILE)


def _deg_body(dst_hbm, ones_hbm, zeros_hbm, out_hbm, idx_v, ones_v, shared):
    c = lax.axis_index("c")
    s = lax.axis_index("s")
    half = CH // 2
    pltpu.sync_copy(dst_hbm.at[s, pl.ds(c * half, half)], idx_v)
    pltpu.sync_copy(ones_hbm, ones_v)
    r0 = s * RPT
    pltpu.sync_copy(zeros_hbm.at[pl.ds(r0, RPT)], shared.at[pl.ds(r0, RPT)])
    plsc.subcore_barrier()

    def body(j, carry):
        pltpu.sync_copy(ones_v, shared.at[idx_v.at[j]], add=True)
        return carry

    lax.fori_loop(0, half, body, 0)
    plsc.subcore_barrier()

    @pl.when(c == 0)
    def _():
        pltpu.sync_copy(shared.at[pl.ds(r0, RPT)], out_hbm.at[0, pl.ds(r0, RPT)])

    @pl.when(c == 1)
    def _():
        pltpu.sync_copy(shared.at[pl.ds(r0, RPT)], out_hbm.at[1, pl.ds(r0, RPT)])


_deg_call = pl.kernel(
    _deg_body,
    out_type=jax.ShapeDtypeStruct((NCORE, NP, 16), jnp.float32),
    mesh=_sc_mesh(),
    scratch_types=[
        pltpu.VMEM((CH // 2, 128), jnp.int32),
        pltpu.VMEM((128, 16), jnp.float32),
        pltpu.VMEM_SHARED((NP, 16), jnp.float32),
    ],
)


def _edge_body(ulo_hbm, uhi_hbm, src_hbm, dst_hbm, zeros_hbm, out_hbm,
               src_v, dst_v, rows_v, shared, sem):
    c = lax.axis_index("c")
    s = lax.axis_index("s")
    pltpu.sync_copy(src_hbm.at[s], src_v)
    pltpu.sync_copy(dst_hbm.at[s], dst_v)
    r0 = s * RPT
    pltpu.sync_copy(zeros_hbm.at[pl.ds(r0, RPT)], shared.at[pl.ds(r0, RPT)])
    plsc.subcore_barrier()

    def make_body(tbl):
        def body(j, carry):
            pltpu.async_copy(tbl.at[src_v.at[j]], rows_v, sem).wait()
            pltpu.sync_copy(rows_v, shared.at[dst_v.at[j]], add=True)
            return carry
        return body

    @pl.when(c == 0)
    def _():
        lax.fori_loop(0, CH, make_body(ulo_hbm), 0)

    @pl.when(c == 1)
    def _():
        lax.fori_loop(0, CH, make_body(uhi_hbm), 0)

    plsc.subcore_barrier()

    @pl.when(c == 0)
    def _():
        pltpu.sync_copy(shared.at[pl.ds(r0, RPT)], out_hbm.at[0, pl.ds(r0, RPT)])

    @pl.when(c == 1)
    def _():
        pltpu.sync_copy(shared.at[pl.ds(r0, RPT)], out_hbm.at[1, pl.ds(r0, RPT)])


_edge_call = pl.kernel(
    _edge_body,
    out_type=jax.ShapeDtypeStruct((NCORE, NP, 64), jnp.float32),
    mesh=_sc_mesh(),
    scratch_types=[
        pltpu.VMEM((CH, 128), jnp.int32),
        pltpu.VMEM((CH, 128), jnp.int32),
        pltpu.VMEM((128, 64), jnp.float32),
        pltpu.VMEM_SHARED((NP, 64), jnp.float32),
        pltpu.SemaphoreType.DMA,
    ],
)


def _tc1_body(x_ref, degp_ref, wl_ref, bl_ref, wr_ref, br_ref, w1_ref,
              ulo_ref, uhi_ref, dinv_ref):
    i = pl.program_id(0)
    deg = degp_ref[0, :, 0:1] + degp_ref[1, :, 0:1] + 1.0
    rows = i * BLK + lax.broadcasted_iota(jnp.int32, (BLK, 1), 0)
    dinv = jnp.where(rows < NS, lax.rsqrt(deg), 0.0)
    x = x_ref[...]
    hl = jnp.dot(x, wl_ref[...], preferred_element_type=jnp.float32,
                 precision=_PREC) + bl_ref[...]
    hr = jnp.dot(x, wr_ref[...], preferred_element_type=jnp.float32,
                 precision=_PREC) + br_ref[...]
    h = jnp.where(rows < NN, hl, hr)
    u = dinv * jnp.dot(h, w1_ref[...], preferred_element_type=jnp.float32,
                       precision=_PREC)
    ulo_ref[...] = u[:, :64]
    uhi_ref[...] = u[:, 64:]
    dinv_ref[...] = dinv


def _tc1(x, degp, wl, bl, wr, br, w1):
    return pl.pallas_call(
        _tc1_body,
        grid=(GRID,),
        in_specs=[
            pl.BlockSpec((BLK, DD), lambda i: (i, 0)),
            pl.BlockSpec((NCORE, BLK, 16), lambda i: (0, i, 0)),
            pl.BlockSpec((DD, DD), lambda i: (0, 0)),
            pl.BlockSpec((1, DD), lambda i: (0, 0)),
            pl.BlockSpec((DD, DD), lambda i: (0, 0)),
            pl.BlockSpec((1, DD), lambda i: (0, 0)),
            pl.BlockSpec((DD, DD), lambda i: (0, 0)),
        ],
        out_specs=[
            pl.BlockSpec((BLK, 64), lambda i: (i, 0)),
            pl.BlockSpec((BLK, 64), lambda i: (i, 0)),
            pl.BlockSpec((BLK, 1), lambda i: (i, 0)),
        ],
        out_shape=[
            jax.ShapeDtypeStruct((NP, 64), jnp.float32),
            jax.ShapeDtypeStruct((NP, 64), jnp.float32),
            jax.ShapeDtypeStruct((NP, 1), jnp.float32),
        ],
    )(x, degp, wl, bl, wr, br, w1)


def _tc2_body(acc_ref, ulo_ref, uhi_ref, dinv_ref, b1_ref, w2_ref,
              u2lo_ref, u2hi_ref):
    u = jnp.concatenate([ulo_ref[...], uhi_ref[...]], axis=1)
    accf = jnp.concatenate([acc_ref[0], acc_ref[1]], axis=1)
    dinv = dinv_ref[...]
    o = jnp.maximum(dinv * (accf + u) + b1_ref[...], 0.0)
    u2 = dinv * jnp.dot(o, w2_ref[...], preferred_element_type=jnp.float32,
                        precision=_PREC)
    u2lo_ref[...] = u2[:, :64]
    u2hi_ref[...] = u2[:, 64:]


def _tc2(acc, ulo, uhi, dinv, b1, w2):
    return pl.pallas_call(
        _tc2_body,
        grid=(GRID,),
        in_specs=[
            pl.BlockSpec((NCORE, BLK, 64), lambda i: (0, i, 0)),
            pl.BlockSpec((BLK, 64), lambda i: (i, 0)),
            pl.BlockSpec((BLK, 64), lambda i: (i, 0)),
            pl.BlockSpec((BLK, 1), lambda i: (i, 0)),
            pl.BlockSpec((1, DD), lambda i: (0, 0)),
            pl.BlockSpec((DD, DD), lambda i: (0, 0)),
        ],
        out_specs=[
            pl.BlockSpec((BLK, 64), lambda i: (i, 0)),
            pl.BlockSpec((BLK, 64), lambda i: (i, 0)),
        ],
        out_shape=[
            jax.ShapeDtypeStruct((NP, 64), jnp.float32),
            jax.ShapeDtypeStruct((NP, 64), jnp.float32),
        ],
    )(acc, ulo, uhi, dinv, b1, w2)


def _tc3_body(acc_ref, ulo_ref, uhi_ref, dinv_ref, b2_ref, seg_ref,
              wfc_ref, bfc_ref, out_ref, psum):
    i = pl.program_id(0)

    @pl.when(i == 0)
    def _():
        psum[...] = jnp.zeros_like(psum)

    u2 = jnp.concatenate([ulo_ref[...], uhi_ref[...]], axis=1)
    accf = jnp.concatenate([acc_ref[0], acc_ref[1]], axis=1)
    dinv = dinv_ref[...]
    z = dinv * (accf + u2) + b2_ref[...]
    seg = seg_ref[...]
    segs = lax.broadcasted_iota(jnp.int32, (BLK, 256), 1)
    onehot = (seg == segs).astype(jnp.float32)
    zaug = jnp.concatenate([z, jnp.ones((BLK, DD), jnp.float32)], axis=1)
    psum[...] += lax.dot_general(
        onehot, zaug, (((0,), (0,)), ((), ())),
        preferred_element_type=jnp.float32, precision=_PREC)

    @pl.when(i == GRID - 1)
    def _():
        ps = psum[...]
        cnt = jnp.maximum(ps[:, DD:DD + 1], 1.0)
        p = ps[:, :DD] / cnt
        out = (jnp.dot(p[0:NGG], wfc_ref[0:DD],
                       preferred_element_type=jnp.float32, precision=_PREC)
               + jnp.dot(p[NGG:2 * NGG], wfc_ref[DD:2 * DD],
                         preferred_element_type=jnp.float32, precision=_PREC)
               + bfc_ref[...])
        out_ref[...] = out


def _tc3(acc, ulo, uhi, dinv, b2, seg, wfc, bfc):
    return pl.pallas_call(
        _tc3_body,
        grid=(GRID,),
        in_specs=[
            pl.BlockSpec((NCORE, BLK, 64), lambda i: (0, i, 0)),
            pl.BlockSpec((BLK, 64), lambda i: (i, 0)),
            pl.BlockSpec((BLK, 64), lambda i: (i, 0)),
            pl.BlockSpec((BLK, 1), lambda i: (i, 0)),
            pl.BlockSpec((1, DD), lambda i: (0, 0)),
            pl.BlockSpec((BLK, 1), lambda i: (i, 0)),
            pl.BlockSpec((2 * DD, 2), lambda i: (0, 0)),
            pl.BlockSpec((1, 2), lambda i: (0, 0)),
        ],
        out_specs=pl.BlockSpec((NGG, 2), lambda i: (0, 0)),
        out_shape=jax.ShapeDtypeStruct((NGG, 2), jnp.float32),
        scratch_shapes=[pltpu.VMEM((256, 256), jnp.float32)],
    )(acc, ulo, uhi, dinv, b2, seg, wfc, bfc)


def kernel(x1, edge_index1, batch1, x2, edge_index2, batch2,
           Wfcl, bfcl, Wfcr, bfcr, W1, b1, W2, b2, Wfc1, bfc1):
    f32 = jnp.float32
    x = jnp.zeros((NP, DD), f32).at[:NN].set(x1).at[NN:NS].set(x2)
    src = (jnp.full((EP,), NS, jnp.int32)
           .at[:EE].set(edge_index1[0])
           .at[EE:ET].set(edge_index2[0] + NN))
    dst = (jnp.full((EP,), NS, jnp.int32)
           .at[:EE].set(edge_index1[1])
           .at[EE:ET].set(edge_index2[1] + NN))
    src_r = src.reshape(NTILE, CH, 128)
    dst_r = dst.reshape(NTILE, CH, 128)
    seg = (jnp.full((NP, 1), 2 * NGG, jnp.int32)
           .at[:NN, 0].set(batch1)
           .at[NN:NS, 0].set(batch2 + NGG))
    zeros64 = jnp.zeros((NP, 64), f32)
    zeros16 = jnp.zeros((NP, 16), f32)
    onescol = jnp.zeros((128, 16), f32).at[:, 0].set(1.0)

    degp = _deg_call(dst_r, onescol, zeros16)
    ulo, uhi, dinv = _tc1(x, degp, Wfcl, bfcl.reshape(1, -1),
                          Wfcr, bfcr.reshape(1, -1), W1)
    acc1 = _edge_call(ulo, uhi, src_r, dst_r, zeros64)
    u2lo, u2hi = _tc2(acc1, ulo, uhi, dinv, b1.reshape(1, -1), W2)
    acc2 = _edge_call(u2lo, u2hi, src_r, dst_r, zeros64)
    out = _tc3(acc2, u2lo, u2hi, dinv, b2.reshape(1, -1), seg,
               Wfc1, bfc1.reshape(1, -1))
    return out


# speedup vs baseline: 11.7744x; 11.7744x over previous
import jax
import jax.numpy as jnp
from jax import lax
from jax.experimental import pallas as pl
from jax.experimental.pallas import tpu as pltpu
from jax.experimental.pallas import tpu_sc as plsc

NN = 10000
EE = 320000
NGG = 100
DD = 128

NPC = 10240
NP = 2 * NPC
NTILE = 16
NCORE = 2
CH = 158
SEG = 2
SCH = CH // SEG
TE = CH * 128
RPT = NPC // NTILE
BLK = 1024
GRID = NP // BLK

_PREC = lax.Precision.HIGHEST


def _sc_mesh():
    return plsc.VectorSubcoreMesh(
        core_axis_name="c", subcore_axis_name="s",
        num_cores=NCORE, num_subcores=NTILE)


def _deg_body(dst_hbm, ones_hbm, zeros_hbm, out_hbm, dst_v, ones_v, shared):
    c = lax.axis_index("c")
    s = lax.axis_index("s")
    pltpu.sync_copy(dst_hbm.at[c, s], dst_v)
    pltpu.sync_copy(ones_hbm, ones_v)
    r0 = s * RPT
    pltpu.sync_copy(zeros_hbm.at[pl.ds(r0, RPT)], shared.at[pl.ds(r0, RPT)])
    plsc.subcore_barrier()

    def body(j, carry):
        pltpu.sync_copy(ones_v, shared.at[dst_v.at[j]], add=True)
        return carry

    lax.fori_loop(0, CH, body, 0)
    plsc.subcore_barrier()

    @pl.when(c == 0)
    def _():
        pltpu.sync_copy(shared.at[pl.ds(r0, RPT)], out_hbm.at[0, pl.ds(r0, RPT)])

    @pl.when(c == 1)
    def _():
        pltpu.sync_copy(shared.at[pl.ds(r0, RPT)], out_hbm.at[1, pl.ds(r0, RPT)])


_deg_call = pl.kernel(
    _deg_body,
    out_type=jax.ShapeDtypeStruct((NCORE, NPC, 16), jnp.float32),
    mesh=_sc_mesh(),
    scratch_types=[
        pltpu.VMEM((CH, 128), jnp.int32),
        pltpu.VMEM((128, 16), jnp.float32),
        pltpu.VMEM_SHARED((NPC, 16), jnp.float32),
    ],
    compiler_params=pltpu.CompilerParams(use_tc_tiling_on_sc=False),
)


def _edge_body(u_hbm, src_hbm, dst_hbm, zeros_hbm, out_hbm,
               src_v, dst_v, rows_v, shared, sem):
    c = lax.axis_index("c")
    s = lax.axis_index("s")
    r0 = s * RPT
    pltpu.sync_copy(zeros_hbm.at[pl.ds(r0, RPT)], shared.at[pl.ds(r0, RPT)])
    plsc.subcore_barrier()

    def seg_body(g, carry):
        pltpu.sync_copy(src_hbm.at[c, s, pl.ds(g * SCH, SCH)], src_v)
        pltpu.sync_copy(dst_hbm.at[c, s, pl.ds(g * SCH, SCH)], dst_v)

        def body(j, carry2):
            pltpu.async_copy(u_hbm.at[src_v.at[j]], rows_v, sem).wait()
            pltpu.sync_copy(rows_v, shared.at[dst_v.at[j]], add=True)
            return carry2

        lax.fori_loop(0, SCH, body, 0)
        return carry

    lax.fori_loop(0, SEG, seg_body, 0)
    plsc.subcore_barrier()

    @pl.when(c == 0)
    def _():
        pltpu.sync_copy(shared.at[pl.ds(r0, RPT)], out_hbm.at[0, pl.ds(r0, RPT)])

    @pl.when(c == 1)
    def _():
        pltpu.sync_copy(shared.at[pl.ds(r0, RPT)], out_hbm.at[1, pl.ds(r0, RPT)])


def _mk_edge_call():
    return pl.kernel(
        _edge_body,
        out_type=jax.ShapeDtypeStruct((NCORE, NPC, DD), jnp.float32),
        mesh=_sc_mesh(),
        scratch_types=[
            pltpu.VMEM((SCH, 128), jnp.int32),
            pltpu.VMEM((SCH, 128), jnp.int32),
            pltpu.VMEM((128, DD), jnp.float32),
            pltpu.VMEM_SHARED((NPC, DD), jnp.float32),
            pltpu.SemaphoreType.DMA,
        ],
        compiler_params=pltpu.CompilerParams(use_tc_tiling_on_sc=False),
    )


_edge_call1 = _mk_edge_call()
_edge_call2 = _mk_edge_call()


def _tc1_body(x_ref, deg_ref, wl_ref, bl_ref, wr_ref, br_ref, w1_ref,
              u_ref, dinv_ref):
    i = pl.program_id(0)
    deg = deg_ref[:, 0:1] + 1.0
    rows = i * BLK + lax.broadcasted_iota(jnp.int32, (BLK, 1), 0)
    valid = (rows < NN) | ((rows >= NPC) & (rows < NPC + NN))
    dinv = jnp.where(valid, lax.rsqrt(deg), 0.0)
    x = x_ref[...]
    hl = jnp.dot(x, wl_ref[...], preferred_element_type=jnp.float32,
                 precision=_PREC) + bl_ref[...]
    hr = jnp.dot(x, wr_ref[...], preferred_element_type=jnp.float32,
                 precision=_PREC) + br_ref[...]
    h = jnp.where(rows < NPC, hl, hr)
    u = dinv * jnp.dot(h, w1_ref[...], preferred_element_type=jnp.float32,
                       precision=_PREC)
    u_ref[...] = u
    dinv_ref[...] = dinv


def _tc1(x, deg, wl, bl, wr, br, w1):
    return pl.pallas_call(
        _tc1_body,
        grid=(GRID,),
        in_specs=[
            pl.BlockSpec((BLK, DD), lambda i: (i, 0)),
            pl.BlockSpec((BLK, 16), lambda i: (i, 0)),
            pl.BlockSpec((DD, DD), lambda i: (0, 0)),
            pl.BlockSpec((1, DD), lambda i: (0, 0)),
            pl.BlockSpec((DD, DD), lambda i: (0, 0)),
            pl.BlockSpec((1, DD), lambda i: (0, 0)),
            pl.BlockSpec((DD, DD), lambda i: (0, 0)),
        ],
        out_specs=[
            pl.BlockSpec((BLK, DD), lambda i: (i, 0)),
            pl.BlockSpec((BLK, 1), lambda i: (i, 0)),
        ],
        out_shape=[
            jax.ShapeDtypeStruct((NP, DD), jnp.float32),
            jax.ShapeDtypeStruct((NP, 1), jnp.float32),
        ],
    )(x, deg, wl, bl, wr, br, w1)


def _tc2_body(acc_ref, u_ref, dinv_ref, b1_ref, w2_ref, u2_ref):
    dinv = dinv_ref[...]
    o = jnp.maximum(dinv * (acc_ref[...] + u_ref[...]) + b1_ref[...], 0.0)
    u2_ref[...] = dinv * jnp.dot(o, w2_ref[...],
                                 preferred_element_type=jnp.float32,
                                 precision=_PREC)


def _tc2(acc, u, dinv, b1, w2):
    return pl.pallas_call(
        _tc2_body,
        grid=(GRID,),
        in_specs=[
            pl.BlockSpec((BLK, DD), lambda i: (i, 0)),
            pl.BlockSpec((BLK, DD), lambda i: (i, 0)),
            pl.BlockSpec((BLK, 1), lambda i: (i, 0)),
            pl.BlockSpec((1, DD), lambda i: (0, 0)),
            pl.BlockSpec((DD, DD), lambda i: (0, 0)),
        ],
        out_specs=pl.BlockSpec((BLK, DD), lambda i: (i, 0)),
        out_shape=jax.ShapeDtypeStruct((NP, DD), jnp.float32),
    )(acc, u, dinv, b1, w2)


def _tc3_body(acc_ref, u2_ref, dinv_ref, b2_ref, seg_ref,
              wfc_ref, bfc_ref, out_ref, psum):
    i = pl.program_id(0)

    @pl.when(i == 0)
    def _():
        psum[...] = jnp.zeros_like(psum)

    dinv = dinv_ref[...]
    z = dinv * (acc_ref[...] + u2_ref[...]) + b2_ref[...]
    seg = seg_ref[...]
    segs = lax.broadcasted_iota(jnp.int32, (BLK, 256), 1)
    onehot = (seg == segs).astype(jnp.float32)
    zaug = jnp.concatenate([z, jnp.ones((BLK, DD), jnp.float32)], axis=1)
    psum[...] += lax.dot_general(
        onehot, zaug, (((0,), (0,)), ((), ())),
        preferred_element_type=jnp.float32, precision=_PREC)

    @pl.when(i == GRID - 1)
    def _():
        ps = psum[...]
        cnt = jnp.maximum(ps[:, DD:DD + 1], 1.0)
        p = ps[:, :DD] / cnt
        out = (jnp.dot(p[0:NGG], wfc_ref[0:DD],
                       preferred_element_type=jnp.float32, precision=_PREC)
               + jnp.dot(p[NGG:2 * NGG], wfc_ref[DD:2 * DD],
                         preferred_element_type=jnp.float32, precision=_PREC)
               + bfc_ref[...])
        out_ref[...] = out


def _tc3(acc, u2, dinv, b2, seg, wfc, bfc):
    return pl.pallas_call(
        _tc3_body,
        grid=(GRID,),
        in_specs=[
            pl.BlockSpec((BLK, DD), lambda i: (i, 0)),
            pl.BlockSpec((BLK, DD), lambda i: (i, 0)),
            pl.BlockSpec((BLK, 1), lambda i: (i, 0)),
            pl.BlockSpec((1, DD), lambda i: (0, 0)),
            pl.BlockSpec((BLK, 1), lambda i: (i, 0)),
            pl.BlockSpec((2 * DD, 2), lambda i: (0, 0)),
            pl.BlockSpec((1, 2), lambda i: (0, 0)),
        ],
        out_specs=pl.BlockSpec((NGG, 2), lambda i: (0, 0)),
        out_shape=jax.ShapeDtypeStruct((NGG, 2), jnp.float32),
        scratch_shapes=[pltpu.VMEM((256, 256), jnp.float32)],
    )(acc, u2, dinv, b2, seg, wfc, bfc)


def _pad_idx(a, fill):
    return (jnp.full((NTILE * TE,), fill, jnp.int32)
            .at[:EE].set(a).reshape(NTILE, CH, 128))


def kernel(x1, edge_index1, batch1, x2, edge_index2, batch2,
           Wfcl, bfcl, Wfcr, bfcr, W1, b1, W2, b2, Wfc1, bfc1):
    f32 = jnp.float32
    x = (jnp.zeros((NP, DD), f32)
         .at[:NN].set(x1).at[NPC:NPC + NN].set(x2))
    srcs = jnp.stack([_pad_idx(edge_index1[0], 0),
                      _pad_idx(edge_index2[0] + NPC, 0)])
    dsts = jnp.stack([_pad_idx(edge_index1[1], NPC - 1),
                      _pad_idx(edge_index2[1], NPC - 1)])
    seg = (jnp.full((NP, 1), 2 * NGG, jnp.int32)
           .at[:NN, 0].set(batch1)
           .at[NPC:NPC + NN, 0].set(batch2 + NGG))
    zeros_acc = jnp.zeros((NPC, DD), f32)
    zeros_deg = jnp.zeros((NPC, 16), f32)
    onescol = jnp.zeros((128, 16), f32).at[:, 0].set(1.0)

    degp = _deg_call(dsts, onescol, zeros_deg)
    deg = degp.reshape(NP, 16)
    u, dinv = _tc1(x, deg, Wfcl, bfcl.reshape(1, -1),
                   Wfcr, bfcr.reshape(1, -1), W1)
    acc1 = _edge_call1(u, srcs, dsts, zeros_acc).reshape(NP, DD)
    u2 = _tc2(acc1, u, dinv, b1.reshape(1, -1), W2)
    acc2 = _edge_call2(u2, srcs, dsts, zeros_acc).reshape(NP, DD)
    out = _tc3(acc2, u2, dinv, b2.reshape(1, -1), seg,
               Wfc1, bfc1.reshape(1, -1))
    return out
